# baseline (speedup 1.0000x reference)
import functools

import jax
import jax.numpy as jnp
from jax import lax
from jax.experimental import pallas as pl
from jax.experimental.pallas import tpu as pltpu
from jax.experimental.pallas import tpu_sc as plsc

_B = 16384
_D = 32
_V = 1000000
_NW = 32
_BPW = _B // _NW
_CH = 128
_NCH = _BPW // _CH

_TPR = 7813
_FULL = _TPR - 1
_KROWS = _D // 8
_RAW_ROWS = _KROWS * _TPR * 8
_RAW_N = _RAW_ROWS * 128


_KC = 28
_NCHK = _FULL // _KC
_JOBS = 2 * _KROWS * _NCHK
_GPW = (_JOBS + _NW - 1) // _NW
_W = _KC * 128


def _clone_body(uemb_hbm, iemb_hbm, utail_hbm, itail_hbm,
                uraw_hbm, iraw_hbm, bin_v, sem_i, sem_o):
    wid = lax.axis_index("s") * 2 + lax.axis_index("c")

    def decode(g):
        j = (wid + g * _NW) % _JOBS
        t = j // (_KROWS * _NCHK)
        r = j - t * (_KROWS * _NCHK)
        k = r // _NCHK
        c = r - k * _NCHK
        return t, k, c

    def issue_in(g, b):
        t, k, c = decode(g)
        src_l = pl.multiple_of(c * _W, 128)
        row = pl.multiple_of(k * 8, 8)

        @pl.when(t == 0)
        def _u():
            pltpu.async_copy(
                uemb_hbm.at[pl.ds(row, 8), pl.ds(src_l, _W)],
                bin_v.at[b], sem_i)

        @pl.when(t == 1)
        def _i():
            pltpu.async_copy(
                iemb_hbm.at[pl.ds(row, 8), pl.ds(src_l, _W)],
                bin_v.at[b], sem_i)

    issue_in(0, 0)

    def drain_out(n):
        def body(d, carry2):
            pltpu.make_async_copy(
                bin_v.at[0, pl.ds(0, 8), pl.ds(0, 128)],
                uraw_hbm.at[pl.ds(0, 8), pl.ds(0, 128)], sem_o).wait()
            return carry2
        lax.fori_loop(0, n, body, 0)

    def step(g, carry):
        b = lax.rem(g, 4)
        pltpu.make_async_copy(
            uemb_hbm.at[pl.ds(0, 8), pl.ds(0, _W)], bin_v.at[0],
            sem_i).wait()
        @pl.when(g >= 3)
        def _dr():
            drain_out(_KC)

        @pl.when(g + 1 < _GPW)
        def _nx():
            issue_in(g + 1, lax.rem(g + 1, 4))

        t, k, c = decode(g)
        dst_row = pl.multiple_of((k * _TPR + c * _KC) * 8, 8)

        @pl.when(t == 0)
        def _ou():
            for t8 in range(_KC):
                pltpu.async_copy(
                    bin_v.at[b, pl.ds(0, 8), pl.ds(t8 * 128, 128)],
                    uraw_hbm.at[pl.ds(dst_row + t8 * 8, 8), pl.ds(0, 128)],
                    sem_o)

        @pl.when(t == 1)
        def _oi():
            for t8 in range(_KC):
                pltpu.async_copy(
                    bin_v.at[b, pl.ds(0, 8), pl.ds(t8 * 128, 128)],
                    iraw_hbm.at[pl.ds(dst_row + t8 * 8, 8), pl.ds(0, 128)],
                    sem_o)
        return carry

    lax.fori_loop(0, _GPW, step, 0)
    drain_out(3 * _KC)

    @pl.when(wid == 0)
    def _tail():
        for src, dst in ((utail_hbm, uraw_hbm), (itail_hbm, iraw_hbm)):
            for k in range(_KROWS):
                row = pl.multiple_of((k * _TPR + _FULL) * 8, 8)
                pltpu.async_copy(
                    src.at[pl.ds(k * 8, 8), pl.ds(0, 128)],
                    dst.at[pl.ds(row, 8), pl.ds(0, 128)], sem_i)
        for _ in range(2 * _KROWS):
            pltpu.make_async_copy(
                utail_hbm.at[pl.ds(0, 8), pl.ds(0, 128)],
                uraw_hbm.at[pl.ds(0, 8), pl.ds(0, 128)], sem_i).wait()


def _gather_body(uid_hbm, iid_hbm, uraw_hbm, iraw_hbm, out_hbm,
                 ubid_v, ibid_v, upre_v, ipre_v, udst_v, idst_v, out_v, sem):
    wid = lax.axis_index("s") * 2 + lax.axis_index("c")
    base = wid * _BPW
    iota16 = lax.iota(jnp.int32, 16)
    del iota16

    def chunk(ch, carry):
        off = ch * _CH
        pltpu.sync_copy(uid_hbm.at[pl.ds(base + off, _CH)], ubid_v)
        pltpu.sync_copy(iid_hbm.at[pl.ds(base + off, _CH)], ibid_v)
        for j in range(_CH // 16):
            for bid, pre in ((ubid_v, upre_v), (ibid_v, ipre_v)):
                i = bid[pl.ds(j * 16, 16)]
                pre[pl.ds(j * 16, 16)] = (
                    (i >> 7) * 1024 + (i & 127))
        copies = []
        for f in range(_D):
            cf = (f // 8) * _TPR * 1024 + (f % 8) * 128
            for pre, raw, dstv in ((upre_v, uraw_hbm, udst_v),
                                   (ipre_v, iraw_hbm, idst_v)):
                copies.append(pltpu.async_copy(
                    raw.at[pl.ds(cf, _RAW_N - cf)].at[pre], dstv.at[f], sem))
        for cp in copies:
            cp.wait()
        for j in range(_CH // 16):
            acc = jnp.zeros((16,), jnp.float32)
            for f in range(_D):
                acc = acc + (udst_v[f, pl.ds(j * 16, 16)] *
                             idst_v[f, pl.ds(j * 16, 16)])
            out_v[pl.ds(off + j * 16, 16)] = 1.0 / (1.0 + jnp.exp(-acc))
        return carry

    lax.fori_loop(0, _NCH, chunk, 0)
    pltpu.sync_copy(out_v, out_hbm.at[pl.ds(base, _BPW)])


@jax.jit
def _run(users_id, items_id, user_emb, item_emb):
    mesh = plsc.VectorSubcoreMesh(core_axis_name="c", subcore_axis_name="s")
    clone = functools.partial(
        pl.kernel,
        mesh=mesh,
        out_type=(jax.ShapeDtypeStruct((_RAW_ROWS, 128), jnp.float32),
                  jax.ShapeDtypeStruct((_RAW_ROWS, 128), jnp.float32)),
        scratch_types=[
            pltpu.VMEM((4, 8, _W), jnp.float32),
            pltpu.SemaphoreType.DMA,
            pltpu.SemaphoreType.DMA,
        ],
        compiler_params=pltpu.CompilerParams(use_tc_tiling_on_sc=True),
    )(_clone_body)
    pad = ((0, 0), (0, 128 - (_V - _FULL * 128)))
    utail = jnp.pad(user_emb.T[:, _FULL * 128:], pad)
    itail = jnp.pad(item_emb.T[:, _FULL * 128:], pad)
    uraw, iraw = clone(user_emb.T, item_emb.T, utail, itail)

    gather = functools.partial(
        pl.kernel,
        mesh=mesh,
        out_type=jax.ShapeDtypeStruct((_B,), jnp.float32),
        scratch_types=[
            pltpu.VMEM((_CH,), jnp.int32),
            pltpu.VMEM((_CH,), jnp.int32),
            pltpu.VMEM((_CH,), jnp.int32),
            pltpu.VMEM((_CH,), jnp.int32),
            pltpu.VMEM((_D, _CH), jnp.float32),
            pltpu.VMEM((_D, _CH), jnp.float32),
            pltpu.VMEM((_BPW,), jnp.float32),
            pltpu.SemaphoreType.DMA,
        ],
        compiler_params=pltpu.CompilerParams(
            needs_layout_passes=False, use_tc_tiling_on_sc=False),
    )(_gather_body)
    return gather(users_id.astype(jnp.int32), items_id.astype(jnp.int32),
                  uraw.reshape(_RAW_N), iraw.reshape(_RAW_N))


def kernel(users_id, items_id, user_emb, item_emb):
    return _run(users_id, items_id, user_emb, item_emb)


# speedup vs baseline: 1.0204x; 1.0204x over previous
import functools

import jax
import jax.numpy as jnp
from jax import lax
from jax.experimental import pallas as pl
from jax.experimental.pallas import tpu as pltpu
from jax.experimental.pallas import tpu_sc as plsc

_B = 16384
_D = 32
_V = 1000000
_NW = 32
_BPW = _B // _NW
_CH = 128
_NCH = _BPW // _CH

_TPR = 7813
_FULL = _TPR - 1
_KROWS = _D // 8
_RAW_ROWS = _KROWS * _TPR * 8
_RAW_N = _RAW_ROWS * 128


_KC = 28
_NCHK = _FULL // _KC
_JOBS = 2 * _KROWS * _NCHK
_GPW = (_JOBS + _NW - 1) // _NW
_W = _KC * 128


def _clone_body(uemb_hbm, iemb_hbm, utail_hbm, itail_hbm,
                uraw_hbm, iraw_hbm, bin_v, sem_i, sem_o):
    wid = lax.axis_index("s") * 2 + lax.axis_index("c")

    def decode(g):
        j = (wid + g * _NW) % _JOBS
        t = j // (_KROWS * _NCHK)
        r = j - t * (_KROWS * _NCHK)
        k = r // _NCHK
        c = r - k * _NCHK
        return t, k, c

    def issue_in(g, b):
        t, k, c = decode(g)
        src_l = pl.multiple_of(c * _W, 128)
        row = pl.multiple_of(k * 8, 8)

        @pl.when(t == 0)
        def _u():
            pltpu.async_copy(
                uemb_hbm.at[pl.ds(row, 8), pl.ds(src_l, _W)],
                bin_v.at[b], sem_i)

        @pl.when(t == 1)
        def _i():
            pltpu.async_copy(
                iemb_hbm.at[pl.ds(row, 8), pl.ds(src_l, _W)],
                bin_v.at[b], sem_i)

    issue_in(0, 0)

    def drain_out(n):
        def body(d, carry2):
            pltpu.make_async_copy(
                bin_v.at[0, pl.ds(0, 8), pl.ds(0, 128)],
                uraw_hbm.at[pl.ds(0, 8), pl.ds(0, 128)], sem_o).wait()
            return carry2
        lax.fori_loop(0, n, body, 0)

    def step(g, carry):
        b = lax.rem(g, 3)
        pltpu.make_async_copy(
            uemb_hbm.at[pl.ds(0, 8), pl.ds(0, _W)], bin_v.at[0],
            sem_i).wait()
        @pl.when(g >= 2)
        def _dr():
            drain_out(_KC)

        @pl.when(g + 1 < _GPW)
        def _nx():
            issue_in(g + 1, lax.rem(g + 1, 3))

        t, k, c = decode(g)
        dst_row = pl.multiple_of((k * _TPR + c * _KC) * 8, 8)

        @pl.when(t == 0)
        def _ou():
            for t8 in range(_KC):
                pltpu.async_copy(
                    bin_v.at[b, pl.ds(0, 8), pl.ds(t8 * 128, 128)],
                    uraw_hbm.at[pl.ds(dst_row + t8 * 8, 8), pl.ds(0, 128)],
                    sem_o)

        @pl.when(t == 1)
        def _oi():
            for t8 in range(_KC):
                pltpu.async_copy(
                    bin_v.at[b, pl.ds(0, 8), pl.ds(t8 * 128, 128)],
                    iraw_hbm.at[pl.ds(dst_row + t8 * 8, 8), pl.ds(0, 128)],
                    sem_o)
        return carry

    lax.fori_loop(0, _GPW, step, 0)
    drain_out(2 * _KC)

    @pl.when(wid == 0)
    def _tail():
        for src, dst in ((utail_hbm, uraw_hbm), (itail_hbm, iraw_hbm)):
            for k in range(_KROWS):
                row = pl.multiple_of((k * _TPR + _FULL) * 8, 8)
                pltpu.async_copy(
                    src.at[pl.ds(k * 8, 8), pl.ds(0, 128)],
                    dst.at[pl.ds(row, 8), pl.ds(0, 128)], sem_i)
        for _ in range(2 * _KROWS):
            pltpu.make_async_copy(
                utail_hbm.at[pl.ds(0, 8), pl.ds(0, 128)],
                uraw_hbm.at[pl.ds(0, 8), pl.ds(0, 128)], sem_i).wait()


def _gather_body(uid_hbm, iid_hbm, uraw_hbm, iraw_hbm, out_hbm,
                 ubid_v, ibid_v, upre_v, ipre_v, udst_v, idst_v, out_v, sem):
    wid = lax.axis_index("s") * 2 + lax.axis_index("c")
    base = wid * _BPW

    pltpu.sync_copy(uid_hbm.at[pl.ds(base, _BPW)], ubid_v)
    pltpu.sync_copy(iid_hbm.at[pl.ds(base, _BPW)], ibid_v)
    for j in range(_BPW // 16):
        for bid, pre in ((ubid_v, upre_v), (ibid_v, ipre_v)):
            i = bid[pl.ds(j * 16, 16)]
            pre[pl.ds(j * 16, 16)] = (i >> 7) * 1024 + (i & 127)

    def issue(ch, b):
        off = ch * _CH
        for f in range(_D):
            cf = (f // 8) * _TPR * 1024 + (f % 8) * 128
            for pre, raw, dstv in ((upre_v, uraw_hbm, udst_v),
                                   (ipre_v, iraw_hbm, idst_v)):
                pltpu.async_copy(
                    raw.at[pl.ds(cf, _RAW_N - cf)].at[
                        pre.at[pl.ds(off, _CH)]],
                    dstv.at[b, f], sem)

    def drain():
        def body(d, carry2):
            pltpu.make_async_copy(
                uraw_hbm.at[pl.ds(0, _CH)], udst_v.at[0, 0], sem).wait()
            return carry2
        lax.fori_loop(0, 2 * _D, body, 0)

    issue(0, 0)

    def chunk(ch, carry):
        b = lax.rem(ch, 2)
        drain()

        @pl.when(ch + 1 < _NCH)
        def _nx():
            issue(ch + 1, 1 - b)

        off = ch * _CH
        for j in range(_CH // 16):
            acc = jnp.zeros((16,), jnp.float32)
            for f in range(_D):
                acc = acc + (udst_v[b, f, pl.ds(j * 16, 16)] *
                             idst_v[b, f, pl.ds(j * 16, 16)])
            out_v[pl.ds(off + j * 16, 16)] = 1.0 / (1.0 + jnp.exp(-acc))
        return carry

    lax.fori_loop(0, _NCH, chunk, 0)
    pltpu.sync_copy(out_v, out_hbm.at[pl.ds(base, _BPW)])


@jax.jit
def _run(users_id, items_id, user_emb, item_emb):
    mesh = plsc.VectorSubcoreMesh(core_axis_name="c", subcore_axis_name="s")
    clone = functools.partial(
        pl.kernel,
        mesh=mesh,
        out_type=(jax.ShapeDtypeStruct((_RAW_ROWS, 128), jnp.float32),
                  jax.ShapeDtypeStruct((_RAW_ROWS, 128), jnp.float32)),
        scratch_types=[
            pltpu.VMEM((3, 8, _W), jnp.float32),
            pltpu.SemaphoreType.DMA,
            pltpu.SemaphoreType.DMA,
        ],
        compiler_params=pltpu.CompilerParams(use_tc_tiling_on_sc=True),
    )(_clone_body)
    pad = ((0, 0), (0, 128 - (_V - _FULL * 128)))
    utail = jnp.pad(user_emb.T[:, _FULL * 128:], pad)
    itail = jnp.pad(item_emb.T[:, _FULL * 128:], pad)
    uraw, iraw = clone(user_emb.T, item_emb.T, utail, itail)

    gather = functools.partial(
        pl.kernel,
        mesh=mesh,
        out_type=jax.ShapeDtypeStruct((_B,), jnp.float32),
        scratch_types=[
            pltpu.VMEM((_BPW,), jnp.int32),
            pltpu.VMEM((_BPW,), jnp.int32),
            pltpu.VMEM((_BPW,), jnp.int32),
            pltpu.VMEM((_BPW,), jnp.int32),
            pltpu.VMEM((2, _D, _CH), jnp.float32),
            pltpu.VMEM((2, _D, _CH), jnp.float32),
            pltpu.VMEM((_BPW,), jnp.float32),
            pltpu.SemaphoreType.DMA,
        ],
        compiler_params=pltpu.CompilerParams(
            needs_layout_passes=False, use_tc_tiling_on_sc=False),
    )(_gather_body)
    return gather(users_id.astype(jnp.int32), items_id.astype(jnp.int32),
                  uraw.reshape(_RAW_N), iraw.reshape(_RAW_N))


def kernel(users_id, items_id, user_emb, item_emb):
    return _run(users_id, items_id, user_emb, item_emb)
